# Initial kernel scaffold; baseline (speedup 1.0000x reference)
import jax
import jax.numpy as jnp
from jax.experimental import pallas as pl


def kernel(x, embedding, offsets, w1, b1, w2, b2, w3, b3, w4, b4):
    raise NotImplementedError("write your pallas kernel here")


# speedup vs baseline: 9.5313x; 9.5313x over previous
import functools

import jax
import jax.numpy as jnp
from jax.experimental import pallas as pl
from jax.experimental.pallas import tpu as pltpu

_VW = 16


def _round_up(x, m):
    return (x + m - 1) // m * m


def _table_kernel(e_ref, w1_ref, o_ref):
    nfvw = e_ref.shape[0]
    d = e_ref.shape[1]
    nf = nfvw // _VW
    for f in range(nf):
        w_blk = w1_ref[f * d:(f + 1) * d, :]
        e_blk = e_ref[f * _VW:(f + 1) * _VW, :]
        blk = jax.lax.dot_general(
            w_blk, e_blk, (((0,), (1,)), ((), ())),
            preferred_element_type=jnp.float32)
        o_ref[:, f * _VW:(f + 1) * _VW] = blk.astype(o_ref.dtype)


def _build_table(e2, w1):
    nfvw, d = e2.shape
    H1 = w1.shape[1]
    return pl.pallas_call(
        _table_kernel,
        out_shape=jax.ShapeDtypeStruct((H1, nfvw), jnp.bfloat16),
    )(e2, w1)


def _mlp_kernel(idx_ref, t_ref, b1_ref, w2_ref, b2_ref, w3_ref, b3_ref,
                w4_ref, b4_ref, o_ref):
    nf, bm = idx_ref.shape

    idx = idx_ref[...]
    iota_v = jax.lax.broadcasted_iota(jnp.int32, (nf, _VW, bm), 1)
    mh = (idx.reshape(nf, 1, bm) == iota_v).astype(jnp.bfloat16)
    mh = mh.reshape(nf * _VW, bm)

    h = jnp.dot(t_ref[...], mh, preferred_element_type=jnp.float32)
    h = jnp.maximum(h + b1_ref[...], 0.0)

    h = jnp.dot(w2_ref[...], h, preferred_element_type=jnp.float32)
    h = jnp.maximum(h + b2_ref[...], 0.0)

    h = jnp.dot(w3_ref[...], h, preferred_element_type=jnp.float32)
    h = jnp.maximum(h + b3_ref[...], 0.0)

    out = jnp.sum(h * w4_ref[...], axis=0, keepdims=True) + b4_ref[...]
    o_ref[...] = out.astype(o_ref.dtype)


def _mlp_call(idxT, t2t, b1T, w2T, b2T, w3T, b3T, w4, b4, *, block_m):
    nf, B_pad = idxT.shape
    H1, nfvw = t2t.shape
    H2 = w2T.shape[0]
    H3 = w3T.shape[0]
    bm = block_m
    grid = (B_pad // bm,)

    full2 = lambda shape: pl.BlockSpec(shape, lambda i: (0, 0))

    flops = 2 * B_pad * (H1 * nfvw + H1 * H2 + H2 * H3 + H3)
    bytes_accessed = (idxT.size * 4 + t2t.size * 2
                      + (w2T.size + w3T.size) * 2
                      + (b1T.size + b2T.size + b3T.size + w4.size + b4.size) * 4
                      + B_pad * 4)

    return pl.pallas_call(
        _mlp_kernel,
        out_shape=jax.ShapeDtypeStruct((1, B_pad), jnp.float32),
        grid=grid,
        in_specs=[
            pl.BlockSpec((nf, bm), lambda i: (0, i)),
            full2((H1, nfvw)),
            full2((H1, 1)),
            full2((H2, H1)), full2((H2, 1)),
            full2((H3, H2)), full2((H3, 1)),
            full2((H3, 1)), full2((1, 1)),
        ],
        out_specs=pl.BlockSpec((1, bm), lambda i: (0, i)),
        compiler_params=pltpu.CompilerParams(
            dimension_semantics=("parallel",)),
        cost_estimate=pl.CostEstimate(
            flops=flops, transcendentals=0, bytes_accessed=bytes_accessed),
    )(idxT, t2t, b1T, w2T, b2T, w3T, b3T, w4, b4)


@jax.jit
def _forward(x, embedding, offsets, w1, b1, w2, b2, w3, b3, w4, b4):
    B, nf = x.shape
    vocab, d = embedding.shape
    H1 = w1.shape[1]

    bm = min(1024, _round_up(B, 128))
    B_pad = _round_up(B, bm)
    if B_pad // bm < 2:
        half = (B_pad // 2) // 128 * 128
        if half >= 128:
            bm = half
            B_pad = _round_up(B, bm)

    idxT = jnp.pad(x.T, ((0, 0), (0, B_pad - B)))

    c = jnp.arange(nf * _VW, dtype=jnp.int32)
    rows = jnp.clip(offsets[c // _VW] + c % _VW, 0, vocab - 1)
    e2 = embedding[rows]

    t2t = _build_table(e2, w1)

    out_row = _mlp_call(
        idxT, t2t,
        b1.T,
        w2.T, b2.T,
        w3.T, b3.T,
        w4, b4,
        block_m=bm)
    return out_row[0, :B].reshape(B, 1)


def kernel(x, embedding, offsets, w1, b1, w2, b2, w3, b3, w4, b4):
    return _forward(x, embedding, offsets, w1, b1, w2, b2, w3, b3, w4, b4)


# speedup vs baseline: 11.9990x; 1.2589x over previous
import functools

import jax
import jax.numpy as jnp
from jax.experimental import pallas as pl
from jax.experimental.pallas import tpu as pltpu

_VW = 16


def _round_up(x, m):
    return (x + m - 1) // m * m


def _table_kernel(e_ref, w1_ref, o_ref):
    nfvw = e_ref.shape[0]
    d = e_ref.shape[1]
    nf = nfvw // _VW
    for f in range(nf):
        w_blk = w1_ref[f * d:(f + 1) * d, :]
        e_blk = e_ref[f * _VW:(f + 1) * _VW, :]
        blk = jax.lax.dot_general(
            w_blk, e_blk, (((0,), (1,)), ((), ())),
            preferred_element_type=jnp.float32)
        o_ref[:, f * _VW:(f + 1) * _VW] = blk.astype(o_ref.dtype)


def _build_table(e2, w1):
    nfvw, d = e2.shape
    H1 = w1.shape[1]
    return pl.pallas_call(
        _table_kernel,
        out_shape=jax.ShapeDtypeStruct((H1, nfvw), jnp.bfloat16),
    )(e2, w1)


def _mlp_kernel(idx_ref, t_ref, b1_ref, w2_ref, b2_ref, w3_ref, b3_ref,
                w4_ref, b4_ref, o_ref):
    nf, bm = idx_ref.shape

    idx = idx_ref[...]
    iota_v = jax.lax.broadcasted_iota(jnp.int32, (nf, _VW, bm), 1)
    mh = (idx.reshape(nf, 1, bm) == iota_v).astype(jnp.bfloat16)
    mh = mh.reshape(nf * _VW, bm)

    h = jnp.dot(t_ref[...], mh, preferred_element_type=jnp.float32)
    h = jnp.maximum(h + b1_ref[...], 0.0)

    h = jnp.dot(w2_ref[...], h, preferred_element_type=jnp.float32)
    h = jnp.maximum(h + b2_ref[...], 0.0)

    h = jnp.dot(w3_ref[...], h, preferred_element_type=jnp.float32)
    h = jnp.maximum(h + b3_ref[...], 0.0)

    out = jnp.sum(h * w4_ref[...], axis=0, keepdims=True) + b4_ref[...]
    o_ref[...] = out.astype(o_ref.dtype)


def _mlp_call(idxT, t2t, b1T, w2T, b2T, w3T, b3T, w4, b4, *, block_m):
    nf, B_pad = idxT.shape
    H1, nfvw = t2t.shape
    H2 = w2T.shape[0]
    H3 = w3T.shape[0]
    bm = block_m
    grid = (B_pad // bm,)

    full2 = lambda shape: pl.BlockSpec(shape, lambda i: (0, 0))

    flops = 2 * B_pad * (H1 * nfvw + H1 * H2 + H2 * H3 + H3)
    bytes_accessed = (idxT.size * 4 + t2t.size * 2
                      + (w2T.size + w3T.size) * 2
                      + (b1T.size + b2T.size + b3T.size + w4.size + b4.size) * 4
                      + B_pad * 4)

    return pl.pallas_call(
        _mlp_kernel,
        out_shape=jax.ShapeDtypeStruct((1, B_pad), jnp.float32),
        grid=grid,
        in_specs=[
            pl.BlockSpec((nf, bm), lambda i: (0, i)),
            full2((H1, nfvw)),
            full2((H1, 1)),
            full2((H2, H1)), full2((H2, 1)),
            full2((H3, H2)), full2((H3, 1)),
            full2((H3, 1)), full2((1, 1)),
        ],
        out_specs=pl.BlockSpec((1, bm), lambda i: (0, i)),
        compiler_params=pltpu.CompilerParams(
            dimension_semantics=("parallel",)),
        cost_estimate=pl.CostEstimate(
            flops=flops, transcendentals=0, bytes_accessed=bytes_accessed),
    )(idxT, t2t, b1T, w2T, b2T, w3T, b3T, w4, b4)


@jax.jit
def _forward(x, embedding, offsets, w1, b1, w2, b2, w3, b3, w4, b4):
    B, nf = x.shape
    vocab, d = embedding.shape
    H1 = w1.shape[1]

    bm = min(2048, _round_up(B, 128))
    B_pad = _round_up(B, bm)
    if B_pad // bm < 2:
        half = (B_pad // 2) // 128 * 128
        if half >= 128:
            bm = half
            B_pad = _round_up(B, bm)

    idxT = jnp.pad(x.T, ((0, 0), (0, B_pad - B)))

    c = jnp.arange(nf * _VW, dtype=jnp.int32)
    rows = jnp.clip(offsets[c // _VW] + c % _VW, 0, vocab - 1)
    e2 = embedding[rows]

    t2t = _build_table(e2, w1)

    out_row = _mlp_call(
        idxT, t2t,
        b1.T,
        w2.T, b2.T,
        w3.T, b3.T,
        w4, b4,
        block_m=bm)
    return out_row[0, :B].reshape(B, 1)


def kernel(x, embedding, offsets, w1, b1, w2, b2, w3, b3, w4, b4):
    return _forward(x, embedding, offsets, w1, b1, w2, b2, w3, b3, w4, b4)


# speedup vs baseline: 13.5046x; 1.1255x over previous
import functools

import jax
import jax.numpy as jnp
from jax.experimental import pallas as pl
from jax.experimental.pallas import tpu as pltpu

_VW = 16


def _round_up(x, m):
    return (x + m - 1) // m * m


def _table_kernel(e_ref, w1_ref, o_ref):
    nfvw = e_ref.shape[0]
    d = e_ref.shape[1]
    nf = nfvw // _VW
    for f in range(nf):
        w_blk = w1_ref[f * d:(f + 1) * d, :]
        e_blk = e_ref[f * _VW:(f + 1) * _VW, :]
        blk = jax.lax.dot_general(
            w_blk, e_blk, (((0,), (1,)), ((), ())),
            preferred_element_type=jnp.float32)
        o_ref[:, f * _VW:(f + 1) * _VW] = blk.astype(o_ref.dtype)


def _build_table(e2, w1):
    nfvw, d = e2.shape
    H1 = w1.shape[1]
    return pl.pallas_call(
        _table_kernel,
        out_shape=jax.ShapeDtypeStruct((H1, nfvw), jnp.bfloat16),
    )(e2, w1)


def _mlp_kernel(idx_ref, t_ref, b1_ref, w2_ref, b2_ref, w3_ref, b3_ref,
                w4_ref, b4_ref, o_ref):
    nf, bm = idx_ref.shape

    idx = idx_ref[...]
    iota_v = jax.lax.broadcasted_iota(jnp.int32, (nf, _VW, bm), 1)
    mh = (idx.reshape(nf, 1, bm) == iota_v).astype(jnp.bfloat16)
    mh = mh.reshape(nf * _VW, bm)

    h = jnp.dot(t_ref[...], mh, preferred_element_type=jnp.float32)
    h = jnp.maximum(h + b1_ref[...], 0.0)

    h = jnp.dot(w2_ref[...], h, preferred_element_type=jnp.float32)
    h = jnp.maximum(h + b2_ref[...], 0.0)

    h = jnp.dot(w3_ref[...], h, preferred_element_type=jnp.float32)
    h = jnp.maximum(h + b3_ref[...], 0.0)

    out = jnp.sum(h * w4_ref[...], axis=0, keepdims=True) + b4_ref[...]
    o_ref[...] = out.astype(o_ref.dtype)


def _mlp_call(idxT, t2t, b1T, w2T, b2T, w3T, b3T, w4, b4, *, block_m):
    nf, B_pad = idxT.shape
    H1, nfvw = t2t.shape
    H2 = w2T.shape[0]
    H3 = w3T.shape[0]
    bm = block_m
    grid = (B_pad // bm,)

    full2 = lambda shape: pl.BlockSpec(shape, lambda i: (0, 0))

    flops = 2 * B_pad * (H1 * nfvw + H1 * H2 + H2 * H3 + H3)
    bytes_accessed = (idxT.size * 4 + t2t.size * 2
                      + (w2T.size + w3T.size) * 2
                      + (b1T.size + b2T.size + b3T.size + w4.size + b4.size) * 4
                      + B_pad * 4)

    return pl.pallas_call(
        _mlp_kernel,
        out_shape=jax.ShapeDtypeStruct((1, B_pad), jnp.float32),
        grid=grid,
        in_specs=[
            pl.BlockSpec((nf, bm), lambda i: (0, i)),
            full2((H1, nfvw)),
            full2((H1, 1)),
            full2((H2, H1)), full2((H2, 1)),
            full2((H3, H2)), full2((H3, 1)),
            full2((H3, 1)), full2((1, 1)),
        ],
        out_specs=pl.BlockSpec((1, bm), lambda i: (0, i)),
        compiler_params=pltpu.CompilerParams(
            dimension_semantics=("parallel",)),
        cost_estimate=pl.CostEstimate(
            flops=flops, transcendentals=0, bytes_accessed=bytes_accessed),
    )(idxT, t2t, b1T, w2T, b2T, w3T, b3T, w4, b4)


@jax.jit
def _forward(x, embedding, offsets, w1, b1, w2, b2, w3, b3, w4, b4):
    B, nf = x.shape
    vocab, d = embedding.shape
    H1 = w1.shape[1]

    bm = min(4096, _round_up(B, 128))
    B_pad = _round_up(B, bm)
    if B_pad // bm < 2:
        half = (B_pad // 2) // 128 * 128
        if half >= 128:
            bm = half
            B_pad = _round_up(B, bm)

    idxT = jnp.pad(x.T, ((0, 0), (0, B_pad - B)))

    c = jnp.arange(nf * _VW, dtype=jnp.int32)
    rows = jnp.clip(offsets[c // _VW] + c % _VW, 0, vocab - 1)
    e2 = embedding[rows]

    t2t = _build_table(e2, w1)

    out_row = _mlp_call(
        idxT, t2t,
        b1.T,
        w2.T, b2.T,
        w3.T, b3.T,
        w4, b4,
        block_m=bm)
    return out_row[0, :B].reshape(B, 1)


def kernel(x, embedding, offsets, w1, b1, w2, b2, w3, b3, w4, b4):
    return _forward(x, embedding, offsets, w1, b1, w2, b2, w3, b3, w4, b4)


# speedup vs baseline: 14.0739x; 1.0422x over previous
import functools

import jax
import jax.numpy as jnp
from jax.experimental import pallas as pl
from jax.experimental.pallas import tpu as pltpu

_VW = 16


def _round_up(x, m):
    return (x + m - 1) // m * m


def _table_kernel(e_ref, w1_ref, o_ref):
    nfvw = e_ref.shape[0]
    d = e_ref.shape[1]
    nf = nfvw // _VW
    for f in range(nf):
        w_blk = w1_ref[f * d:(f + 1) * d, :]
        e_blk = e_ref[f * _VW:(f + 1) * _VW, :]
        blk = jax.lax.dot_general(
            w_blk, e_blk, (((0,), (1,)), ((), ())),
            preferred_element_type=jnp.float32)
        o_ref[:, f * _VW:(f + 1) * _VW] = blk.astype(o_ref.dtype)


def _build_table(e2, w1):
    nfvw, d = e2.shape
    H1 = w1.shape[1]
    return pl.pallas_call(
        _table_kernel,
        out_shape=jax.ShapeDtypeStruct((H1, nfvw), jnp.bfloat16),
    )(e2, w1)


def _mlp_kernel(idx_ref, t_ref, b1_ref, w2_ref, b2_ref, w3_ref, b3_ref,
                w4_ref, b4_ref, o_ref):
    nf, bm = idx_ref.shape

    idx = idx_ref[...]
    iota_v = jax.lax.broadcasted_iota(jnp.int32, (nf, _VW, bm), 1)
    mh = (idx.reshape(nf, 1, bm) == iota_v).astype(jnp.bfloat16)
    mh = mh.reshape(nf * _VW, bm)

    h = jnp.dot(t_ref[...], mh, preferred_element_type=jnp.float32)
    h = jnp.maximum(h + b1_ref[...], 0.0)

    h = jnp.dot(w2_ref[...], h, preferred_element_type=jnp.float32)
    h = jnp.maximum(h + b2_ref[...], 0.0)

    h = jnp.dot(w3_ref[...], h, preferred_element_type=jnp.float32)
    h = jnp.maximum(h + b3_ref[...], 0.0)

    out = jnp.sum(h * w4_ref[...], axis=0, keepdims=True) + b4_ref[...]
    o_ref[...] = out.astype(o_ref.dtype)


def _mlp_call(idxT, t2t, b1T, w2T, b2T, w3T, b3T, w4, b4, *, block_m):
    nf, B_pad = idxT.shape
    H1, nfvw = t2t.shape
    H2 = w2T.shape[0]
    H3 = w3T.shape[0]
    bm = block_m
    grid = (B_pad // bm,)

    full2 = lambda shape: pl.BlockSpec(shape, lambda i: (0, 0))

    flops = 2 * B_pad * (H1 * nfvw + H1 * H2 + H2 * H3 + H3)
    bytes_accessed = (idxT.size * 4 + t2t.size * 2
                      + (w2T.size + w3T.size) * 2
                      + (b1T.size + b2T.size + b3T.size + w4.size + b4.size) * 4
                      + B_pad * 4)

    return pl.pallas_call(
        _mlp_kernel,
        out_shape=jax.ShapeDtypeStruct((1, B_pad), jnp.float32),
        grid=grid,
        in_specs=[
            pl.BlockSpec((nf, bm), lambda i: (0, i)),
            full2((H1, nfvw)),
            full2((H1, 1)),
            full2((H2, H1)), full2((H2, 1)),
            full2((H3, H2)), full2((H3, 1)),
            full2((H3, 1)), full2((1, 1)),
        ],
        out_specs=pl.BlockSpec((1, bm), lambda i: (0, i)),
        compiler_params=pltpu.CompilerParams(
            dimension_semantics=("parallel",)),
        cost_estimate=pl.CostEstimate(
            flops=flops, transcendentals=0, bytes_accessed=bytes_accessed),
    )(idxT, t2t, b1T, w2T, b2T, w3T, b3T, w4, b4)


@jax.jit
def _forward(x, embedding, offsets, w1, b1, w2, b2, w3, b3, w4, b4):
    B, nf = x.shape
    vocab, d = embedding.shape
    H1 = w1.shape[1]

    bm = min(8192, _round_up(B, 128))
    B_pad = _round_up(B, bm)
    if B_pad // bm < 2:
        half = (B_pad // 2) // 128 * 128
        if half >= 128:
            bm = half
            B_pad = _round_up(B, bm)

    idxT = jnp.pad(x.T, ((0, 0), (0, B_pad - B)))

    c = jnp.arange(nf * _VW, dtype=jnp.int32)
    rows = jnp.clip(offsets[c // _VW] + c % _VW, 0, vocab - 1)
    e2 = embedding[rows]

    t2t = _build_table(e2, w1)

    out_row = _mlp_call(
        idxT, t2t,
        b1.T,
        w2.T, b2.T,
        w3.T, b3.T,
        w4, b4,
        block_m=bm)
    return out_row[0, :B].reshape(B, 1)


def kernel(x, embedding, offsets, w1, b1, w2, b2, w3, b3, w4, b4):
    return _forward(x, embedding, offsets, w1, b1, w2, b2, w3, b3, w4, b4)
